# Initial kernel scaffold; baseline (speedup 1.0000x reference)
import jax
import jax.numpy as jnp
from jax.experimental import pallas as pl


def kernel(X, mask, Wn, bn, We, be, gn, gnb, ge, geb):
    raise NotImplementedError("write your pallas kernel here")


# speedup vs baseline: 1.0530x; 1.0530x over previous
import functools

import jax
import jax.numpy as jnp
import numpy as np
from jax.experimental import pallas as pl

_B, _L, _K = 2, 2048, 30
_NUM_PE = 16
_NUM_RBF = 16
_KPAD = 32
_ROWS = 256


def _topk_body(xr_ref, xc_ref, dn_ref, ei_ref):
    xr = xr_ref[0]
    xi = xr[:, 0:1]
    yi = xr[:, 1:2]
    zi = xr[:, 2:3]
    xc = xc_ref[0]
    xj = xc[0:1, :]
    yj = xc[1:2, :]
    zj = xc[2:3, :]
    dx = xi - xj
    dy = yi - yj
    dz = zi - zj
    D = jnp.sqrt(dx * dx + dy * dy + dz * dz + 1e-6)
    lane = jax.lax.broadcasted_iota(jnp.int32, D.shape, 1)
    kiota = jax.lax.broadcasted_iota(jnp.int32, (_ROWS, _KPAD), 1)
    vals = jnp.zeros((_ROWS, _KPAD), jnp.float32)
    idxs = jnp.zeros((_ROWS, _KPAD), jnp.int32)
    work = D
    for t in range(_K):
        m = jnp.min(work, axis=1, keepdims=True)
        sel = jnp.where(work == m, lane, _L)
        am = jnp.min(sel, axis=1, keepdims=True)
        vals = jnp.where(kiota == t, m, vals)
        idxs = jnp.where(kiota == t, am, idxs)
        work = jnp.where(lane == am, jnp.float32(np.inf), work)
    dn_ref[0] = vals
    ei_ref[0] = idxs


def _topk_pallas(Xca):
    xr = jnp.pad(Xca, ((0, 0), (0, 0), (0, 1)))
    xc = jnp.pad(jnp.swapaxes(Xca, 1, 2), ((0, 0), (0, 5), (0, 0)))
    grid = (_B, _L // _ROWS)
    return pl.pallas_call(
        _topk_body,
        grid=grid,
        in_specs=[
            pl.BlockSpec((1, _ROWS, 4), lambda b, r: (b, r, 0)),
            pl.BlockSpec((1, 8, _L), lambda b, r: (b, 0, 0)),
        ],
        out_specs=[
            pl.BlockSpec((1, _ROWS, _KPAD), lambda b, r: (b, r, 0)),
            pl.BlockSpec((1, _ROWS, _KPAD), lambda b, r: (b, r, 0)),
        ],
        out_shape=[
            jax.ShapeDtypeStruct((_B, _L, _KPAD), jnp.float32),
            jax.ShapeDtypeStruct((_B, _L, _KPAD), jnp.int32),
        ],
    )(xr, xc)


def _normalize(x, eps=1e-12):
    n = jnp.linalg.norm(x, axis=-1, keepdims=True)
    return x / jnp.maximum(n, eps)


def _gather_nodes(nodes, E_idx):
    b, l, k = E_idx.shape
    c = nodes.shape[-1]
    idx = E_idx.reshape(b, l * k)
    idx = jnp.broadcast_to(idx[..., None], (b, l * k, c))
    out = jnp.take_along_axis(nodes, idx, axis=1)
    return out.reshape(b, l, k, c)


def _rbf(D):
    D_mu = jnp.linspace(0.0, 20.0, _NUM_RBF).reshape(1, 1, 1, -1)
    D_sigma = 20.0 / _NUM_RBF
    return jnp.exp(-(((D[..., None] - D_mu) / D_sigma) ** 2))


def _pos_embeddings(E_idx):
    n_nodes = E_idx.shape[1]
    ii = jnp.arange(n_nodes, dtype=jnp.float32).reshape(1, -1, 1)
    d = (E_idx.astype(jnp.float32) - ii)[..., None]
    frequency = jnp.exp(
        jnp.arange(0, _NUM_PE, 2, dtype=jnp.float32) * -(np.log(10000.0) / _NUM_PE)
    )
    angles = d * frequency.reshape(1, 1, 1, -1)
    return jnp.concatenate([jnp.cos(angles), jnp.sin(angles)], axis=-1)


def _dihedrals(X, eps=1e-7):
    b, l = X.shape[0], X.shape[1]
    Xb = X[:, :, :3, :].reshape(b, 3 * l, 3)
    dX = Xb[:, 1:, :] - Xb[:, :-1, :]
    U = _normalize(dX)
    u_2 = U[:, :-2, :]
    u_1 = U[:, 1:-1, :]
    u_0 = U[:, 2:, :]
    n_2 = _normalize(jnp.cross(u_2, u_1))
    n_1 = _normalize(jnp.cross(u_1, u_0))
    cosD = jnp.sum(n_2 * n_1, axis=-1)
    cosD = jnp.clip(cosD, -1.0 + eps, 1.0 - eps)
    Dih = jnp.sign(jnp.sum(u_2 * n_1, axis=-1)) * jnp.arccos(cosD)
    Dih = jnp.pad(Dih, ((0, 0), (1, 2)))
    Dih = Dih.reshape(b, l, 3)
    return jnp.concatenate([jnp.cos(Dih), jnp.sin(Dih)], axis=-1)


def _quaternions(R, eps=1e-10):
    diag = jnp.diagonal(R, axis1=-2, axis2=-1)
    Rxx, Ryy, Rzz = diag[..., 0], diag[..., 1], diag[..., 2]
    magnitudes = 0.5 * jnp.sqrt(
        jnp.abs(
            1.0
            + jnp.stack([Rxx - Ryy - Rzz, -Rxx + Ryy - Rzz, -Rxx - Ryy + Rzz], axis=-1)
            + eps
        )
    )

    def _R(i, j):
        return R[..., i, j]

    signs = jnp.sign(
        jnp.stack(
            [_R(2, 1) - _R(1, 2), _R(0, 2) - _R(2, 0), _R(1, 0) - _R(0, 1)], axis=-1
        )
    )
    xyz = signs * magnitudes
    w = jnp.sqrt(jax.nn.relu(1.0 + jnp.sum(diag, axis=-1, keepdims=True))) / 2.0
    Q = jnp.concatenate([xyz, w], axis=-1)
    return _normalize(Q)


def _orientations_coarse(Xca, E_idx, eps=1e-6):
    b, l = Xca.shape[0], Xca.shape[1]
    k = E_idx.shape[2]
    dX = Xca[:, 1:, :] - Xca[:, :-1, :]
    U = _normalize(dX)
    u_2 = U[:, :-2, :]
    u_1 = U[:, 1:-1, :]
    n_2 = _normalize(jnp.cross(u_2, u_1))
    o_1 = _normalize(u_2 - u_1)
    O = jnp.stack([o_1, n_2, jnp.cross(o_1, n_2)], axis=2)
    O = O.reshape(b, O.shape[1], 9)
    O = jnp.pad(O, ((0, 0), (1, 2), (0, 0)))
    O_neighbors = _gather_nodes(O, E_idx)
    X_neighbors = _gather_nodes(Xca, E_idx)
    Omat = O.reshape(b, l, 3, 3)
    On = O_neighbors.reshape(b, l, k, 3, 3)
    dXn = X_neighbors - Xca[:, :, None, :]
    dU = jnp.matmul(Omat[:, :, None, :, :], dXn[..., None])[..., 0]
    dU = _normalize(dU)
    Rmat = jnp.matmul(jnp.swapaxes(Omat[:, :, None, :, :], -1, -2), On)
    Q = _quaternions(Rmat)
    return jnp.concatenate([dU, Q], axis=-1)


def _layer_norm(x, g, b, eps=1e-5):
    mu = jnp.mean(x, axis=-1, keepdims=True)
    var = jnp.mean((x - mu) ** 2, axis=-1, keepdims=True)
    return (x - mu) / jnp.sqrt(var + eps) * g + b


def kernel(X, mask, Wn, bn, We, be, gn, gnb, ge, geb):
    Xca = X[:, :, 1, :]
    Dn, Ei = _topk_pallas(Xca)
    D_neighbors = Dn[..., :_K]
    E_idx = Ei[..., :_K]
    RBF = _rbf(D_neighbors)
    E_positional = _pos_embeddings(E_idx)
    V = _dihedrals(X)
    O_features = _orientations_coarse(Xca, E_idx)
    E = jnp.concatenate([E_positional, RBF, O_features], axis=-1)
    V = _layer_norm(V @ Wn + bn, gn, gnb)
    E = _layer_norm(E @ We + be, ge, geb)
    return V, E, E_idx
